# baseline (speedup 1.0000x reference)
import functools

import jax
import jax.numpy as jnp
from jax import lax
from jax.experimental import pallas as pl
from jax.experimental.pallas import tpu as pltpu
from jax.experimental.pallas import tpu_sc as plsc

Z_LOSS_COEF = 0.001
AUX_LOSS_COEF = 0.01

G = 4
T = 8192
E = 64
K = 2

NC = 2
NS = 16
NW = NC * NS
TOK_PER_W = (G * T) // NW
IDX_PER_W = TOK_PER_W * K
ROWS = IDX_PER_W // 128

BT = 2048
NB = T // BT


def _sc_hist_body(idx_hbm, out_hbm, idx_raw, scat_idx, scat_val, zbuf,
                  hist_sh, sem):
    c = lax.axis_index("c")
    s = lax.axis_index("s")
    wid = c * NS + s
    pltpu.sync_copy(idx_hbm.at[pl.ds(wid * ROWS, ROWS)], idx_raw)

    gbase = (wid // (NW // G)) * E
    lane = lax.iota(jnp.int32, 16)
    odd = (lane % 2) == 1
    perm = lane ^ 1

    def row(j, carry):
        for l in range(8):
            w = idx_raw[j, pl.ds(l * 16, 16)]
            partner = lax.gather(
                w, perm[:, None],
                lax.GatherDimensionNumbers(
                    offset_dims=(), collapsed_slice_dims=(0,),
                    start_index_map=(0,)),
                slice_sizes=(1,),
                mode=lax.GatherScatterMode.PROMISE_IN_BOUNDS)
            dup = odd & (w == partner)
            scat_idx[j, pl.ds(l * 16, 16)] = w + gbase
            scat_val[j, pl.ds(l * 16, 16)] = jnp.where(dup, 0.0, 1.0)
        return carry

    lax.fori_loop(0, ROWS, row, 0)

    @pl.when(s == 0)
    def _():
        for i in range(G * E // 16):
            zbuf[pl.ds(i * 16, 16)] = jnp.zeros((16,), jnp.float32)

    @pl.when(s == 0)
    def _():
        pltpu.sync_copy(zbuf, hist_sh)

    plsc.subcore_barrier()
    copies = [
        pltpu.async_copy(scat_val.at[j], hist_sh.at[scat_idx.at[j]],
                         sem, add=True)
        for j in range(ROWS)
    ]
    for h in copies:
        h.wait()
    plsc.subcore_barrier()

    @pl.when(s == 0)
    def _():
        pltpu.sync_copy(hist_sh, out_hbm.at[c])


def _sc_hist(idx_flat):
    mesh = plsc.VectorSubcoreMesh(core_axis_name="c", subcore_axis_name="s")
    fn = functools.partial(
        pl.kernel,
        mesh=mesh,
        out_type=jax.ShapeDtypeStruct((NC, G * E), jnp.float32),
        scratch_types=[
            pltpu.VMEM((ROWS, 128), jnp.int32),
            pltpu.VMEM((ROWS, 128), jnp.int32),
            pltpu.VMEM((ROWS, 128), jnp.float32),
            pltpu.VMEM((G * E,), jnp.float32),
            pltpu.VMEM_SHARED((G * E,), jnp.float32),
            pltpu.SemaphoreType.DMA,
        ],
    )(_sc_hist_body)
    return fn(idx_flat)


def _tc_body(logits_ref, cnt_ref, out_ref, psum_ref, z_ref, aux_ref):
    g = pl.program_id(0)
    b = pl.program_id(1)

    @pl.when((g == 0) & (b == 0))
    def _():
        z_ref[0, 0] = 0.0
        aux_ref[0, 0] = 0.0

    @pl.when(b == 0)
    def _():
        psum_ref[...] = jnp.zeros_like(psum_ref)

    x = logits_ref[0]
    m = jnp.max(x, axis=1, keepdims=True)
    e = jnp.exp(x - m)
    s = jnp.sum(e, axis=1, keepdims=True)
    logz = m + jnp.log(s)
    z_ref[0, 0] += jnp.sum(logz * logz)
    psum_ref[...] += jnp.sum(e * (1.0 / s), axis=0, keepdims=True)

    @pl.when(b == NB - 1)
    def _():
        cnt_g = cnt_ref[pl.ds(g, 1), :] + cnt_ref[pl.ds(g + G, 1), :]
        aux_ref[0, 0] += jnp.sum(cnt_g * psum_ref[...])

    @pl.when((g == G - 1) & (b == NB - 1))
    def _():
        z_loss = z_ref[0, 0] / (G * T)
        aux_loss = aux_ref[0, 0] * E / (T * T * G)
        out_ref[...] = jnp.full(
            (1, 1), Z_LOSS_COEF * z_loss + AUX_LOSS_COEF * aux_loss,
            jnp.float32)


def _tc_main(router_logits, cnt):
    return pl.pallas_call(
        _tc_body,
        grid=(G, NB),
        in_specs=[
            pl.BlockSpec((1, BT, E), lambda g, b: (g, b, 0)),
            pl.BlockSpec((NC * G, E), lambda g, b: (0, 0)),
        ],
        out_specs=pl.BlockSpec((1, 1), lambda g, b: (0, 0)),
        out_shape=jax.ShapeDtypeStruct((1, 1), jnp.float32),
        scratch_shapes=[
            pltpu.VMEM((1, E), jnp.float32),
            pltpu.SMEM((1, 1), jnp.float32),
            pltpu.SMEM((1, 1), jnp.float32),
        ],
    )(router_logits, cnt)


def kernel(router_logits, expert_indexes):
    idx_flat = jnp.reshape(expert_indexes.astype(jnp.int32),
                           (NW * ROWS, 128))
    cnt = _sc_hist(idx_flat)
    cnt = jnp.reshape(cnt, (NC * G, E))
    out = _tc_main(router_logits, cnt)
    return out[0, 0]


# speedup vs baseline: 1.4584x; 1.4584x over previous
import jax
import jax.numpy as jnp
from jax import lax
from jax.experimental import pallas as pl
from jax.experimental.pallas import tpu as pltpu

Z_LOSS_COEF = 0.001
AUX_LOSS_COEF = 0.01
G, T, E, K = 4, 8192, 64, 2
BT = 2048
NB = T // BT


def _tc_body(logits_ref, idx_ref, out_ref, psum_ref, csum_ref, z_ref, aux_ref):
    g = pl.program_id(0)
    b = pl.program_id(1)

    @pl.when((g == 0) & (b == 0))
    def _():
        z_ref[0, 0] = 0.0
        aux_ref[0, 0] = 0.0

    @pl.when(b == 0)
    def _():
        psum_ref[...] = jnp.zeros_like(psum_ref)
        csum_ref[...] = jnp.zeros_like(csum_ref)

    x = logits_ref[0]
    m = jnp.max(x, axis=1, keepdims=True)
    e = jnp.exp(x - m)
    s = jnp.sum(e, axis=1, keepdims=True)
    logz = m + jnp.log(s)
    z_ref[0, 0] += jnp.sum(logz * logz)
    psum_ref[...] += jnp.sum(e * (1.0 / s), axis=0, keepdims=True)

    idx = idx_ref[0]
    bi = lax.broadcasted_iota(jnp.int32, (BT, E), 1)
    m0 = bi == idx[:, 0:1]
    m1 = bi == idx[:, 1:2]
    cnt_blk = (m0 | m1).astype(jnp.float32)
    csum_ref[...] += jnp.sum(cnt_blk, axis=0, keepdims=True)

    @pl.when(b == NB - 1)
    def _():
        aux_ref[0, 0] += jnp.sum(csum_ref[...] * psum_ref[...])

    @pl.when((g == G - 1) & (b == NB - 1))
    def _():
        z_loss = z_ref[0, 0] / (G * T)
        aux_loss = aux_ref[0, 0] * E / (T * T * G)
        out_ref[...] = jnp.full(
            (1, 1), Z_LOSS_COEF * z_loss + AUX_LOSS_COEF * aux_loss,
            jnp.float32)


def kernel(router_logits, expert_indexes):
    out = pl.pallas_call(
        _tc_body,
        grid=(G, NB),
        in_specs=[
            pl.BlockSpec((1, BT, E), lambda g, b: (g, b, 0)),
            pl.BlockSpec((1, BT, K), lambda g, b: (g, b, 0)),
        ],
        out_specs=pl.BlockSpec((1, 1), lambda g, b: (0, 0)),
        out_shape=jax.ShapeDtypeStruct((1, 1), jnp.float32),
        scratch_shapes=[
            pltpu.VMEM((1, E), jnp.float32),
            pltpu.VMEM((1, E), jnp.float32),
            pltpu.SMEM((1, 1), jnp.float32),
            pltpu.SMEM((1, 1), jnp.float32),
        ],
    )(router_logits, expert_indexes.astype(jnp.int32))
    return out[0, 0]


# speedup vs baseline: 1.5145x; 1.0385x over previous
import jax
import jax.numpy as jnp
from jax import lax
from jax.experimental import pallas as pl
from jax.experimental.pallas import tpu as pltpu

Z_LOSS_COEF = 0.001
AUX_LOSS_COEF = 0.01
G, T, E, K = 4, 8192, 64, 2
BT = 8192
NB = T // BT


def _tc_body(logits_ref, idx_ref, out_ref, psum_ref, csum_ref, z_ref, aux_ref):
    g = pl.program_id(0)
    b = pl.program_id(1)

    @pl.when((g == 0) & (b == 0))
    def _():
        z_ref[0, 0] = 0.0
        aux_ref[0, 0] = 0.0

    @pl.when(b == 0)
    def _():
        psum_ref[...] = jnp.zeros_like(psum_ref)
        csum_ref[...] = jnp.zeros_like(csum_ref)

    x = logits_ref[0]
    m = jnp.max(x, axis=1, keepdims=True)
    e = jnp.exp(x - m)
    s = jnp.sum(e, axis=1, keepdims=True)
    logz = m + jnp.log(s)
    z_ref[0, 0] += jnp.sum(logz * logz)
    psum_ref[...] += jnp.sum(e * (1.0 / s), axis=0, keepdims=True)

    idx = idx_ref[0]
    bi = lax.broadcasted_iota(jnp.int32, (BT, E), 1)
    m0 = bi == idx[:, 0:1]
    m1 = bi == idx[:, 1:2]
    cnt_blk = (m0 | m1).astype(jnp.float32)
    csum_ref[...] += jnp.sum(cnt_blk, axis=0, keepdims=True)

    @pl.when(b == NB - 1)
    def _():
        aux_ref[0, 0] += jnp.sum(csum_ref[...] * psum_ref[...])

    @pl.when((g == G - 1) & (b == NB - 1))
    def _():
        z_loss = z_ref[0, 0] / (G * T)
        aux_loss = aux_ref[0, 0] * E / (T * T * G)
        out_ref[...] = jnp.full(
            (1, 1), Z_LOSS_COEF * z_loss + AUX_LOSS_COEF * aux_loss,
            jnp.float32)


def kernel(router_logits, expert_indexes):
    out = pl.pallas_call(
        _tc_body,
        grid=(G, NB),
        in_specs=[
            pl.BlockSpec((1, BT, E), lambda g, b: (g, b, 0)),
            pl.BlockSpec((1, BT, K), lambda g, b: (g, b, 0)),
        ],
        out_specs=pl.BlockSpec((1, 1), lambda g, b: (0, 0)),
        out_shape=jax.ShapeDtypeStruct((1, 1), jnp.float32),
        scratch_shapes=[
            pltpu.VMEM((1, E), jnp.float32),
            pltpu.VMEM((1, E), jnp.float32),
            pltpu.SMEM((1, 1), jnp.float32),
            pltpu.SMEM((1, 1), jnp.float32),
        ],
    )(router_logits, expert_indexes.astype(jnp.int32))
    return out[0, 0]
